# baseline (speedup 1.0000x reference)
import jax
import jax.numpy as jnp
from jax.experimental import pallas as pl
from jax.experimental.pallas import tpu as pltpu

_NUM_FEATURES = 256
_NUM_BATCHES = 8
_N_POINTS = 65536
_EPS = 1e-05
_R = 8192
_NBLK = _N_POINTS // _R
_K = 3
_SUB = 8
_RS = _R // _SUB


def _onehot(idx_ref, j):
    idx = idx_ref[0, :, pl.ds(j * _RS, _RS)]
    iota = jax.lax.broadcasted_iota(jnp.int32, (_NUM_BATCHES, _RS), 0)
    return (iota == idx).astype(jnp.float32)


def _fused_body(x_ref, idx_ref, w_ref, b_ref, mw_ref, vw_ref, o_ref,
                cache_ref, sums_ref, sumsq_ref, cnt_ref,
                scale_ref, shift_ref):
    i = pl.program_id(0)
    j = pl.program_id(1)

    @pl.when(i < _NBLK)
    def _phase1():
        xb = x_ref[pl.ds(j * _RS, _RS), :]
        onehot = _onehot(idx_ref, j)
        dn = (((1,), (0,)), ((), ()))
        s = jax.lax.dot_general(onehot, xb, dn,
                                preferred_element_type=jnp.float32)
        sq = jax.lax.dot_general(onehot, xb * xb, dn,
                                 preferred_element_type=jnp.float32)
        c = jnp.broadcast_to(jnp.sum(onehot, axis=1, keepdims=True),
                             (_NUM_BATCHES, 128))

        @pl.when(jnp.logical_and(i == 0, j == 0))
        def _init():
            sums_ref[...] = s
            sumsq_ref[...] = sq
            cnt_ref[...] = c

        @pl.when(jnp.logical_or(i != 0, j != 0))
        def _acc():
            sums_ref[...] += s
            sumsq_ref[...] += sq
            cnt_ref[...] += c

        @pl.when(i < _K)
        def _store_cache():
            cache_ref[i, pl.ds(j * _RS, _RS), :] = xb

        @pl.when(jnp.logical_and(i == _NBLK - 1, j == _SUB - 1))
        def _finalize():
            cnt = cnt_ref[:, 0:1]
            cs = jnp.maximum(cnt, 1.0)
            sums = sums_ref[...]
            sumsq = sumsq_ref[...]
            mean_in = sums / cs
            ex2 = sumsq / cs
            var_in = ex2 - mean_in * mean_in
            mean_ln = jnp.mean(mean_in, axis=1, keepdims=True)
            var_ln = jnp.mean(ex2, axis=1, keepdims=True) - mean_ln * mean_ln
            tot_s = jnp.sum(sums, axis=0, keepdims=True)
            tot_sq = jnp.sum(sumsq, axis=0, keepdims=True)
            n = jnp.float32(_N_POINTS)
            mean_bn = tot_s / n
            var_bn = (tot_sq - n * mean_bn * mean_bn) / (n - 1.0)

            mw = mw_ref[...]
            mw = jnp.exp(mw - jnp.max(mw, axis=1, keepdims=True))
            mw = mw / jnp.sum(mw, axis=1, keepdims=True)
            vw = vw_ref[...]
            vw = jnp.exp(vw - jnp.max(vw, axis=1, keepdims=True))
            vw = vw / jnp.sum(vw, axis=1, keepdims=True)

            mean = (mw[:, 0:1] * mean_in + mw[:, 1:2] * mean_ln
                    + mw[:, 2:3] * mean_bn)
            var = (vw[:, 0:1] * var_in + vw[:, 1:2] * var_ln
                   + vw[:, 2:3] * var_bn)
            inv = jax.lax.rsqrt(var + _EPS)
            scale_ref[...] = inv * w_ref[...]
            shift_ref[...] = b_ref[...] - mean * (inv * w_ref[...])

    @pl.when(i >= _NBLK)
    def _phase2():
        onehot = _onehot(idx_ref, j)
        dn = (((0,), (0,)), ((), ()))
        g_scale = jax.lax.dot_general(onehot, scale_ref[...], dn,
                                      preferred_element_type=jnp.float32)
        g_shift = jax.lax.dot_general(onehot, shift_ref[...], dn,
                                      preferred_element_type=jnp.float32)
        use_cache = jnp.logical_and(i >= _NBLK + 1, i < _NBLK + 1 + _K)

        @pl.when(use_cache)
        def _from_cache():
            xb = cache_ref[i - (_NBLK + 1), pl.ds(j * _RS, _RS), :]
            o_ref[pl.ds(j * _RS, _RS), :] = xb * g_scale + g_shift

        @pl.when(jnp.logical_not(use_cache))
        def _from_stream():
            xb = x_ref[pl.ds(j * _RS, _RS), :]
            o_ref[pl.ds(j * _RS, _RS), :] = xb * g_scale + g_shift


def _x_imap(i, j):
    blk = jnp.where(i < _NBLK, i,
                    jnp.where(i <= _NBLK + _K, _NBLK - 1, i - (_NBLK + 1)))
    return (blk, 0)


def _idx_imap(i, j):
    blk = jnp.where(i <= _NBLK, jnp.where(i < _NBLK, i, _NBLK - 1),
                    i - (_NBLK + 1))
    return (blk, 0, 0)


def _out_imap(i, j):
    blk = jnp.where(i <= _NBLK, _NBLK - 1, i - (_NBLK + 1))
    return (blk, 0)


def kernel(x, weight, bias, mean_weight, var_weight, batch_indices):
    idx3 = batch_indices.reshape(_NBLK, 1, _R)
    mw2 = mean_weight.reshape(1, 3)
    vw2 = var_weight.reshape(1, 3)

    full = lambda shape: pl.BlockSpec(
        shape, lambda i, j: tuple(0 for _ in shape))

    out = pl.pallas_call(
        _fused_body,
        grid=(2 * _NBLK, _SUB),
        in_specs=[
            pl.BlockSpec((_R, _NUM_FEATURES), _x_imap),
            pl.BlockSpec((1, 1, _R), _idx_imap),
            full((1, _NUM_FEATURES)), full((1, _NUM_FEATURES)),
            full((1, 3)), full((1, 3)),
        ],
        out_specs=pl.BlockSpec((_R, _NUM_FEATURES), _out_imap),
        out_shape=jax.ShapeDtypeStruct((_N_POINTS, _NUM_FEATURES),
                                       jnp.float32),
        scratch_shapes=[
            pltpu.VMEM((_K, _R, _NUM_FEATURES), jnp.float32),
            pltpu.VMEM((_NUM_BATCHES, _NUM_FEATURES), jnp.float32),
            pltpu.VMEM((_NUM_BATCHES, _NUM_FEATURES), jnp.float32),
            pltpu.VMEM((_NUM_BATCHES, 128), jnp.float32),
            pltpu.VMEM((_NUM_BATCHES, _NUM_FEATURES), jnp.float32),
            pltpu.VMEM((_NUM_BATCHES, _NUM_FEATURES), jnp.float32),
        ],
    )(x, idx3, weight, bias, mw2, vw2)
    return out


# speedup vs baseline: 1.8873x; 1.8873x over previous
import jax
import jax.numpy as jnp
from jax.experimental import pallas as pl
from jax.experimental.pallas import tpu as pltpu

_NUM_FEATURES = 256
_NUM_BATCHES = 8
_N_POINTS = 65536
_EPS = 1e-05
_R = 8192
_NBLK = _N_POINTS // _R
_K = 0
_SUB = 1
_RS = _R // _SUB


def _onehot(idx_ref, j):
    idx = idx_ref[0, :, pl.ds(j * _RS, _RS)]
    iota = jax.lax.broadcasted_iota(jnp.int32, (_NUM_BATCHES, _RS), 0)
    return (iota == idx).astype(jnp.float32)


def _fused_body(x_ref, idx_ref, w_ref, b_ref, mw_ref, vw_ref, o_ref,
                cache_ref, sums_ref, sumsq_ref, cnt_ref,
                scale_ref, shift_ref):
    i = pl.program_id(0)
    j = pl.program_id(1)

    @pl.when(i < _NBLK)
    def _phase1():
        xb = x_ref[pl.ds(j * _RS, _RS), :]
        onehot = _onehot(idx_ref, j)
        dn = (((1,), (0,)), ((), ()))
        s = jax.lax.dot_general(onehot, xb, dn,
                                preferred_element_type=jnp.float32)
        sq = jax.lax.dot_general(onehot, xb * xb, dn,
                                 preferred_element_type=jnp.float32)
        c = jnp.broadcast_to(jnp.sum(onehot, axis=1, keepdims=True),
                             (_NUM_BATCHES, 128))

        @pl.when(jnp.logical_and(i == 0, j == 0))
        def _init():
            sums_ref[...] = s
            sumsq_ref[...] = sq
            cnt_ref[...] = c

        @pl.when(jnp.logical_or(i != 0, j != 0))
        def _acc():
            sums_ref[...] += s
            sumsq_ref[...] += sq
            cnt_ref[...] += c

        @pl.when(i < _K)
        def _store_cache():
            cache_ref[i, pl.ds(j * _RS, _RS), :] = xb

        @pl.when(jnp.logical_and(i == _NBLK - 1, j == _SUB - 1))
        def _finalize():
            cnt = cnt_ref[:, 0:1]
            cs = jnp.maximum(cnt, 1.0)
            sums = sums_ref[...]
            sumsq = sumsq_ref[...]
            mean_in = sums / cs
            ex2 = sumsq / cs
            var_in = ex2 - mean_in * mean_in
            mean_ln = jnp.mean(mean_in, axis=1, keepdims=True)
            var_ln = jnp.mean(ex2, axis=1, keepdims=True) - mean_ln * mean_ln
            tot_s = jnp.sum(sums, axis=0, keepdims=True)
            tot_sq = jnp.sum(sumsq, axis=0, keepdims=True)
            n = jnp.float32(_N_POINTS)
            mean_bn = tot_s / n
            var_bn = (tot_sq - n * mean_bn * mean_bn) / (n - 1.0)

            mw = mw_ref[...]
            mw = jnp.exp(mw - jnp.max(mw, axis=1, keepdims=True))
            mw = mw / jnp.sum(mw, axis=1, keepdims=True)
            vw = vw_ref[...]
            vw = jnp.exp(vw - jnp.max(vw, axis=1, keepdims=True))
            vw = vw / jnp.sum(vw, axis=1, keepdims=True)

            mean = (mw[:, 0:1] * mean_in + mw[:, 1:2] * mean_ln
                    + mw[:, 2:3] * mean_bn)
            var = (vw[:, 0:1] * var_in + vw[:, 1:2] * var_ln
                   + vw[:, 2:3] * var_bn)
            inv = jax.lax.rsqrt(var + _EPS)
            scale_ref[...] = inv * w_ref[...]
            shift_ref[...] = b_ref[...] - mean * (inv * w_ref[...])

    @pl.when(i >= _NBLK)
    def _phase2():
        onehot = _onehot(idx_ref, j)
        dn = (((0,), (0,)), ((), ()))
        g_scale = jax.lax.dot_general(onehot, scale_ref[...], dn,
                                      preferred_element_type=jnp.float32)
        g_shift = jax.lax.dot_general(onehot, shift_ref[...], dn,
                                      preferred_element_type=jnp.float32)
        use_cache = jnp.logical_and(i >= _NBLK + 1, i < _NBLK + 1 + _K)

        @pl.when(use_cache)
        def _from_cache():
            xb = cache_ref[i - (_NBLK + 1), pl.ds(j * _RS, _RS), :]
            o_ref[pl.ds(j * _RS, _RS), :] = xb * g_scale + g_shift

        @pl.when(jnp.logical_not(use_cache))
        def _from_stream():
            xb = x_ref[pl.ds(j * _RS, _RS), :]
            o_ref[pl.ds(j * _RS, _RS), :] = xb * g_scale + g_shift


def _x_imap(i, j):
    blk = jnp.where(i < _NBLK, i,
                    jnp.where(i <= _NBLK + _K, _NBLK - 1, i - (_NBLK + 1)))
    return (blk, 0)


def _idx_imap(i, j):
    blk = jnp.where(i <= _NBLK, jnp.where(i < _NBLK, i, _NBLK - 1),
                    i - (_NBLK + 1))
    return (blk, 0, 0)


def _out_imap(i, j):
    blk = jnp.where(i <= _NBLK, _NBLK - 1, i - (_NBLK + 1))
    return (blk, 0)


def kernel(x, weight, bias, mean_weight, var_weight, batch_indices):
    idx3 = batch_indices.reshape(_NBLK, 1, _R)
    mw2 = mean_weight.reshape(1, 3)
    vw2 = var_weight.reshape(1, 3)

    full = lambda shape: pl.BlockSpec(
        shape, lambda i, j: tuple(0 for _ in shape))

    out = pl.pallas_call(
        _fused_body,
        grid=(2 * _NBLK, _SUB),
        in_specs=[
            pl.BlockSpec((_R, _NUM_FEATURES), _x_imap),
            pl.BlockSpec((1, 1, _R), _idx_imap),
            full((1, _NUM_FEATURES)), full((1, _NUM_FEATURES)),
            full((1, 3)), full((1, 3)),
        ],
        out_specs=pl.BlockSpec((_R, _NUM_FEATURES), _out_imap),
        out_shape=jax.ShapeDtypeStruct((_N_POINTS, _NUM_FEATURES),
                                       jnp.float32),
        scratch_shapes=[
            pltpu.VMEM((max(_K, 1), _R, _NUM_FEATURES), jnp.float32),
            pltpu.VMEM((_NUM_BATCHES, _NUM_FEATURES), jnp.float32),
            pltpu.VMEM((_NUM_BATCHES, _NUM_FEATURES), jnp.float32),
            pltpu.VMEM((_NUM_BATCHES, 128), jnp.float32),
            pltpu.VMEM((_NUM_BATCHES, _NUM_FEATURES), jnp.float32),
            pltpu.VMEM((_NUM_BATCHES, _NUM_FEATURES), jnp.float32),
        ],
    )(x, idx3, weight, bias, mw2, vw2)
    return out
